# baseline (speedup 1.0000x reference)
import functools

import jax
import jax.numpy as jnp
from jax.experimental import pallas as pl
from jax.experimental.pallas import tpu as pltpu

TOPK = 2
NEXP = 8


def _moe_body(x_ref, wg_ref, w1_ref, w2_ref,
              out_ref, lb_ref, rz_ref, cnt_ref, w_scr):
    e = pl.program_id(0)

    @pl.when(e == 0)
    def _router():
        xf = x_ref[...]
        logits = jnp.dot(xf, wg_ref[...], preferred_element_type=jnp.float32)
        m = jnp.max(logits, axis=-1, keepdims=True)
        ex = jnp.exp(logits - m)
        ssum = jnp.sum(ex, axis=-1, keepdims=True)
        scores = ex / ssum
        rz = jnp.log(ssum) + m
        rz_ref[0, 0] = jnp.mean(rz * rz)

        col = jax.lax.broadcasted_iota(jnp.int32, scores.shape, 1)
        m1 = jnp.max(scores, axis=-1, keepdims=True)
        idx1 = jnp.min(jnp.where(scores == m1, col, NEXP), axis=-1,
                       keepdims=True)
        sel1 = col == idx1
        s_masked = jnp.where(sel1, -jnp.inf, scores)
        m2 = jnp.max(s_masked, axis=-1, keepdims=True)
        idx2 = jnp.min(jnp.where(s_masked == m2, col, NEXP), axis=-1,
                       keepdims=True)
        sel2 = col == idx2
        picked = sel1 | sel2
        w = jnp.where(picked, scores, 0.0)
        w_scr[...] = w

        counts = jnp.sum(picked.astype(jnp.int32), axis=0)
        cnt_ref[...] = counts[None, :]
        seg_sum = jnp.sum(w, axis=0)
        total = jnp.float32(w.shape[0] * TOPK)
        dist = counts.astype(jnp.float32) / total
        avg = seg_sum / jnp.maximum(counts.astype(jnp.float32), 1.0)
        lb_ref[0, 0] = jnp.sum(dist * avg) * NEXP

    wall = w_scr[...]
    ecol = jax.lax.broadcasted_iota(jnp.int32, wall.shape, 1)
    we = jnp.sum(jnp.where(ecol == e, wall, 0.0), axis=1, keepdims=True)
    h = jnp.maximum(
        jnp.dot(x_ref[...].astype(jnp.bfloat16),
                w1_ref[0].astype(jnp.bfloat16),
                preferred_element_type=jnp.float32),
        0.0)
    contrib = jnp.dot(h.astype(jnp.bfloat16),
                      w2_ref[0].astype(jnp.bfloat16),
                      preferred_element_type=jnp.float32)

    @pl.when(e == 0)
    def _init():
        out_ref[...] = we * contrib

    @pl.when(e > 0)
    def _acc():
        out_ref[...] += we * contrib


@functools.partial(jax.jit, static_argnames=())
def kernel(x, Wg, W1, W2):
    B, S, D = x.shape
    E = W1.shape[0]
    T = B * S
    xf = x.reshape(T, D)

    out, lb, rz, cnt = pl.pallas_call(
        _moe_body,
        grid=(E,),
        in_specs=[
            pl.BlockSpec((T, D), lambda e: (0, 0)),
            pl.BlockSpec((D, E), lambda e: (0, 0)),
            pl.BlockSpec((1, D, W1.shape[2]), lambda e: (e, 0, 0)),
            pl.BlockSpec((1, W2.shape[1], D), lambda e: (e, 0, 0)),
        ],
        out_specs=[
            pl.BlockSpec((T, D), lambda e: (0, 0)),
            pl.BlockSpec(memory_space=pltpu.SMEM),
            pl.BlockSpec(memory_space=pltpu.SMEM),
            pl.BlockSpec((1, E), lambda e: (0, 0)),
        ],
        out_shape=[
            jax.ShapeDtypeStruct((T, D), jnp.float32),
            jax.ShapeDtypeStruct((1, 1), jnp.float32),
            jax.ShapeDtypeStruct((1, 1), jnp.float32),
            jax.ShapeDtypeStruct((1, E), jnp.int32),
        ],
        scratch_shapes=[pltpu.VMEM((T, E), jnp.float32)],
    )(xf, Wg, W1, W2)

    return (out.reshape(B, S, D), lb.reshape(()), rz.reshape(()),
            cnt.reshape(E))


# speedup vs baseline: 1.0335x; 1.0335x over previous
import functools

import jax
import jax.numpy as jnp
from jax.experimental import pallas as pl
from jax.experimental.pallas import tpu as pltpu

TOPK = 2
NEXP = 8
NSLOT = 4


def _moe_body(x_ref, wg_ref, w1_hbm, w2_hbm,
              out_ref, lb_ref, rz_ref, cnt_ref,
              w1v, w2v, w_scr, sem1, sem2):
    for s in range(NSLOT):
        pltpu.make_async_copy(w1_hbm.at[s], w1v.at[s], sem1.at[s]).start()
        pltpu.make_async_copy(w2_hbm.at[s], w2v.at[s], sem2.at[s]).start()

    xf = x_ref[...]
    logits = jnp.dot(xf, wg_ref[...], preferred_element_type=jnp.float32)
    m = jnp.max(logits, axis=-1, keepdims=True)
    ex = jnp.exp(logits - m)
    ssum = jnp.sum(ex, axis=-1, keepdims=True)
    scores = ex / ssum
    rz = jnp.log(ssum) + m
    rz_ref[0, 0] = jnp.mean(rz * rz)

    col = jax.lax.broadcasted_iota(jnp.int32, scores.shape, 1)
    m1 = jnp.max(scores, axis=-1, keepdims=True)
    idx1 = jnp.min(jnp.where(scores == m1, col, NEXP), axis=-1, keepdims=True)
    sel1 = col == idx1
    s_masked = jnp.where(sel1, -jnp.inf, scores)
    m2 = jnp.max(s_masked, axis=-1, keepdims=True)
    idx2 = jnp.min(jnp.where(s_masked == m2, col, NEXP), axis=-1,
                   keepdims=True)
    sel2 = col == idx2
    picked = sel1 | sel2
    w = jnp.where(picked, scores, 0.0)
    w_scr[...] = w

    counts = jnp.sum(picked.astype(jnp.int32), axis=0)
    cnt_ref[...] = counts[None, :]
    seg_sum = jnp.sum(w, axis=0)
    total = jnp.float32(w.shape[0] * TOPK)
    dist = counts.astype(jnp.float32) / total
    avg = seg_sum / jnp.maximum(counts.astype(jnp.float32), 1.0)
    lb_ref[0, 0] = jnp.sum(dist * avg) * NEXP

    xb = xf.astype(jnp.bfloat16)
    wall = w_scr[...]
    ecol = jax.lax.broadcasted_iota(jnp.int32, wall.shape, 1)

    def step(e, _):
        slot = jax.lax.rem(e, NSLOT)
        pltpu.make_async_copy(w1_hbm.at[e], w1v.at[slot], sem1.at[slot]).wait()
        pltpu.make_async_copy(w2_hbm.at[e], w2v.at[slot], sem2.at[slot]).wait()

        we = jnp.sum(jnp.where(ecol == e, wall, 0.0), axis=1, keepdims=True)
        h = jnp.maximum(
            jnp.dot(xb, w1v[slot].astype(jnp.bfloat16),
                    preferred_element_type=jnp.float32), 0.0)
        contrib = jnp.dot(h.astype(jnp.bfloat16),
                          w2v[slot].astype(jnp.bfloat16),
                          preferred_element_type=jnp.float32)

        @pl.when(e == 0)
        def _init():
            out_ref[...] = we * contrib

        @pl.when(e > 0)
        def _acc():
            out_ref[...] += we * contrib

        nxt = e + NSLOT

        @pl.when(nxt < NEXP)
        def _prefetch():
            pltpu.make_async_copy(w1_hbm.at[nxt], w1v.at[slot],
                                  sem1.at[slot]).start()
            pltpu.make_async_copy(w2_hbm.at[nxt], w2v.at[slot],
                                  sem2.at[slot]).start()

        return 0

    jax.lax.fori_loop(0, NEXP, step, 0)


@functools.partial(jax.jit, static_argnames=())
def kernel(x, Wg, W1, W2):
    B, S, D = x.shape
    E = W1.shape[0]
    F = W1.shape[2]
    T = B * S
    xf = x.reshape(T, D)

    out, lb, rz, cnt = pl.pallas_call(
        _moe_body,
        in_specs=[
            pl.BlockSpec((T, D), lambda: (0, 0)),
            pl.BlockSpec((D, E), lambda: (0, 0)),
            pl.BlockSpec(memory_space=pl.ANY),
            pl.BlockSpec(memory_space=pl.ANY),
        ],
        out_specs=[
            pl.BlockSpec((T, D), lambda: (0, 0)),
            pl.BlockSpec(memory_space=pltpu.SMEM),
            pl.BlockSpec(memory_space=pltpu.SMEM),
            pl.BlockSpec((1, E), lambda: (0, 0)),
        ],
        out_shape=[
            jax.ShapeDtypeStruct((T, D), jnp.float32),
            jax.ShapeDtypeStruct((1, 1), jnp.float32),
            jax.ShapeDtypeStruct((1, 1), jnp.float32),
            jax.ShapeDtypeStruct((1, E), jnp.int32),
        ],
        scratch_shapes=[
            pltpu.VMEM((NSLOT, D, F), jnp.float32),
            pltpu.VMEM((NSLOT, F, D), jnp.float32),
            pltpu.VMEM((T, NEXP), jnp.float32),
            pltpu.SemaphoreType.DMA((NSLOT,)),
            pltpu.SemaphoreType.DMA((NSLOT,)),
        ],
    )(xf, Wg, W1, W2)

    return (out.reshape(B, S, D), lb.reshape(()), rz.reshape(()),
            cnt.reshape(E))


# speedup vs baseline: 1.1092x; 1.0733x over previous
import functools

import jax
import jax.numpy as jnp
from jax.experimental import pallas as pl
from jax.experimental.pallas import tpu as pltpu

TOPK = 2
NEXP = 8
NSLOT = 4


def _moe_body(x_ref, wg_ref, w1_hbm, w2_hbm,
              out_ref, lb_ref, rz_ref, cnt_ref,
              w1v, w2v, w_scr, sem1, sem2):
    for s in range(NSLOT):
        pltpu.make_async_copy(w1_hbm.at[s], w1v.at[s], sem1.at[s]).start()
        pltpu.make_async_copy(w2_hbm.at[s], w2v.at[s], sem2.at[s]).start()

    xf = x_ref[...]
    logits = jnp.dot(xf, wg_ref[...], preferred_element_type=jnp.float32)
    m = jnp.max(logits, axis=-1, keepdims=True)
    ex = jnp.exp(logits - m)
    ssum = jnp.sum(ex, axis=-1, keepdims=True)
    scores = ex / ssum
    rz = jnp.log(ssum) + m
    rz_ref[0, 0] = jnp.mean(rz * rz)

    col = jax.lax.broadcasted_iota(jnp.int32, scores.shape, 1)
    m1 = jnp.max(scores, axis=-1, keepdims=True)
    idx1 = jnp.min(jnp.where(scores == m1, col, NEXP), axis=-1, keepdims=True)
    sel1 = col == idx1
    s_masked = jnp.where(sel1, -jnp.inf, scores)
    m2 = jnp.max(s_masked, axis=-1, keepdims=True)
    idx2 = jnp.min(jnp.where(s_masked == m2, col, NEXP), axis=-1,
                   keepdims=True)
    sel2 = col == idx2
    picked = sel1 | sel2
    w = jnp.where(picked, scores, 0.0)
    w_scr[...] = w

    counts = jnp.sum(picked.astype(jnp.int32), axis=0)
    cnt_ref[...] = counts[None, :]
    seg_sum = jnp.sum(w, axis=0)
    total = jnp.float32(w.shape[0] * TOPK)
    dist = counts.astype(jnp.float32) / total
    avg = seg_sum / jnp.maximum(counts.astype(jnp.float32), 1.0)
    lb_ref[0, 0] = jnp.sum(dist * avg) * NEXP

    xb = xf.astype(jnp.bfloat16)
    wall = w_scr[...]
    ecol = jax.lax.broadcasted_iota(jnp.int32, wall.shape, 1)

    for e in range(NEXP):
        slot = e % NSLOT
        pltpu.make_async_copy(w1_hbm.at[e], w1v.at[slot], sem1.at[slot]).wait()
        pltpu.make_async_copy(w2_hbm.at[e], w2v.at[slot], sem2.at[slot]).wait()

        we = jnp.sum(jnp.where(ecol == e, wall, 0.0), axis=1, keepdims=True)
        h = jnp.maximum(
            jnp.dot(xb, w1v[slot].astype(jnp.bfloat16),
                    preferred_element_type=jnp.float32), 0.0)
        contrib = jnp.dot(h.astype(jnp.bfloat16),
                          w2v[slot].astype(jnp.bfloat16),
                          preferred_element_type=jnp.float32)

        if e == 0:
            out_ref[...] = we * contrib
        else:
            out_ref[...] += we * contrib

        nxt = e + NSLOT
        if nxt < NEXP:
            pltpu.make_async_copy(w1_hbm.at[nxt], w1v.at[slot],
                                  sem1.at[slot]).start()
            pltpu.make_async_copy(w2_hbm.at[nxt], w2v.at[slot],
                                  sem2.at[slot]).start()


@functools.partial(jax.jit, static_argnames=())
def kernel(x, Wg, W1, W2):
    B, S, D = x.shape
    E = W1.shape[0]
    F = W1.shape[2]
    T = B * S
    xf = x.reshape(T, D)

    out, lb, rz, cnt = pl.pallas_call(
        _moe_body,
        in_specs=[
            pl.BlockSpec((T, D), lambda: (0, 0)),
            pl.BlockSpec((D, E), lambda: (0, 0)),
            pl.BlockSpec(memory_space=pl.ANY),
            pl.BlockSpec(memory_space=pl.ANY),
        ],
        out_specs=[
            pl.BlockSpec((T, D), lambda: (0, 0)),
            pl.BlockSpec(memory_space=pltpu.SMEM),
            pl.BlockSpec(memory_space=pltpu.SMEM),
            pl.BlockSpec((1, E), lambda: (0, 0)),
        ],
        out_shape=[
            jax.ShapeDtypeStruct((T, D), jnp.float32),
            jax.ShapeDtypeStruct((1, 1), jnp.float32),
            jax.ShapeDtypeStruct((1, 1), jnp.float32),
            jax.ShapeDtypeStruct((1, E), jnp.int32),
        ],
        scratch_shapes=[
            pltpu.VMEM((NSLOT, D, F), jnp.float32),
            pltpu.VMEM((NSLOT, F, D), jnp.float32),
            pltpu.VMEM((T, NEXP), jnp.float32),
            pltpu.SemaphoreType.DMA((NSLOT,)),
            pltpu.SemaphoreType.DMA((NSLOT,)),
        ],
    )(xf, Wg, W1, W2)

    return (out.reshape(B, S, D), lb.reshape(()), rz.reshape(()),
            cnt.reshape(E))
